# baseline (speedup 1.0000x reference)
import functools

import jax
import jax.numpy as jnp
from jax import lax
from jax.experimental import pallas as pl
from jax.experimental.pallas import tpu as pltpu
from jax.experimental.pallas import tpu_sc as plsc

VOCAB = 1000
VPAD = 1024
D = 1000
B, T = 1024, 200
N = B * T

NC, NS, L = 2, 16, 16
NW = NC * NS
NBLK = D // 8
SCHUNK = 4096
NSC = N // SCHUNK
B_PER_W = N // NW


def _row_lse_body(w_ref, out_ref):
    w = w_ref[...]
    m = jnp.max(w, axis=1)
    s = jnp.sum(jnp.exp(w - m[:, None]), axis=1)
    out_ref[...] = jnp.log(s) + m


def _row_lse(w_pad):
    return pl.pallas_call(
        _row_lse_body,
        out_shape=jax.ShapeDtypeStruct((VPAD,), jnp.float32),
    )(w_pad)


_MESH = plsc.VectorSubcoreMesh(core_axis_name="c", subcore_axis_name="s")


@functools.partial(
    pl.kernel,
    mesh=_MESH,
    compiler_params=pltpu.CompilerParams(needs_layout_passes=False),
    out_type=[
        jax.ShapeDtypeStruct((D, N), jnp.float32),
        jax.ShapeDtypeStruct((NW * L,), jnp.float32),
    ],
    scratch_types=[
        pltpu.VMEM((B_PER_W,), jnp.int32),
        pltpu.VMEM((VPAD,), jnp.float32),
        pltpu.VMEM((8, VPAD), jnp.float32),
        pltpu.VMEM((2, SCHUNK), jnp.int32),
        pltpu.VMEM((2, SCHUNK), jnp.int32),
        pltpu.VMEM((2, 8, SCHUNK), jnp.float32),
        pltpu.SemaphoreType.DMA,
        pltpu.SemaphoreType.DMA,
        pltpu.SemaphoreType.DMA,
        pltpu.SemaphoreType.DMA,
    ],
)
def _sc_gather(x_hbm, t_hbm, lse_hbm, wt_hbm, out_hbm, psum_hbm,
               idxl_v, lse_v, wt_v, xv_v, tv_v, tbuf_v,
               p0sem, p1sem, w0sem, w1sem):
    wid = lax.axis_index("s") * NC + lax.axis_index("c")
    psem = (p0sem, p1sem)
    wsem = (w0sem, w1sem)
    lanes = lax.iota(jnp.int32, L)

    pltpu.sync_copy(x_hbm.at[pl.ds(wid * B_PER_W, B_PER_W)], idxl_v)
    pltpu.sync_copy(lse_hbm, lse_v)

    def lse_step(k, acc):
        xv = idxl_v[pl.ds(k * L, L)]
        return acc + plsc.load_gather(lse_v, [xv])

    acc_lse = lax.fori_loop(
        0, B_PER_W // L, lse_step, jnp.zeros((L,), jnp.float32)
    )
    acc = (acc_lse, jnp.zeros((L,), jnp.float32))

    def pf(sc, slot, sem):
        off = sc * SCHUNK
        pltpu.async_copy(x_hbm.at[pl.ds(off, SCHUNK)], xv_v.at[slot], sem)
        pltpu.async_copy(t_hbm.at[pl.ds(off, SCHUNK)], tv_v.at[slot], sem)

    pf(0, 0, psem[0])
    pf(1, 1, psem[1])

    nblk_mine = jnp.where(wid < (NBLK % NW), NBLK // NW + 1, NBLK // NW)

    def block_step(bi, acc):
        blk = bi * NW + wid
        pltpu.sync_copy(wt_hbm.at[pl.ds(blk * 8, 8)], wt_v)
        blk8 = jnp.full((L,), 0, jnp.int32) + blk * 8
        rfill = [jnp.full((L,), r, jnp.int32) for r in range(8)]

        def group_step(g, acc2):
            for b in range(2):
                sc = g * 2 + b
                @pl.when(g >= 1)
                def _():
                    pltpu.make_async_copy(
                        tbuf_v.at[b],
                        out_hbm.at[pl.ds(blk * 8, 8), pl.ds(0, SCHUNK)],
                        wsem[b],
                    ).wait()
                pltpu.make_async_copy(
                    x_hbm.at[pl.ds(0, SCHUNK)], xv_v.at[b], psem[b]
                ).wait()
                pltpu.make_async_copy(
                    t_hbm.at[pl.ds(0, SCHUNK)], tv_v.at[b], psem[b]
                ).wait()

                def k_step(k, acc3):
                    accu = list(acc3)
                    for u in range(2):
                        kk = k * 2 + u
                        xvv = xv_v[b, pl.ds(kk * L, L)]
                        tvv = tv_v[b, pl.ds(kk * L, L)]
                        for r in range(8):
                            v = plsc.load_gather(wt_v, [rfill[r], xvv])
                            tbuf_v[b, r, pl.ds(kk * L, L)] = v
                        tloc = tvv - blk8
                        m = (tloc >= 0) & (tloc < 8)
                        tcl = jnp.clip(tloc, 0, 7)
                        p = plsc.load_gather(wt_v, [tcl, xvv])
                        accu[u] = accu[u] - jnp.where(m, p, 0.0)
                    return tuple(accu)

                acc2 = lax.fori_loop(0, SCHUNK // L // 2, k_step, acc2)
                pltpu.async_copy(
                    tbuf_v.at[b],
                    out_hbm.at[pl.ds(blk * 8, 8), pl.ds(sc * SCHUNK, SCHUNK)],
                    wsem[b],
                )
                pf(lax.rem(sc + 2, NSC), b, psem[b])
            return acc2

        acc = lax.fori_loop(0, NSC // 2, group_step, acc)
        for b in range(2):
            pltpu.make_async_copy(
                tbuf_v.at[b],
                out_hbm.at[pl.ds(blk * 8, 8), pl.ds(0, SCHUNK)],
                wsem[b],
            ).wait()
        return acc

    acc = lax.fori_loop(0, nblk_mine, block_step, acc)

    for b in range(2):
        pltpu.make_async_copy(
            x_hbm.at[pl.ds(0, SCHUNK)], xv_v.at[b], psem[b]
        ).wait()
        pltpu.make_async_copy(
            t_hbm.at[pl.ds(0, SCHUNK)], tv_v.at[b], psem[b]
        ).wait()
    lse_v[pl.ds(0, L)] = acc[0] + acc[1]
    pltpu.sync_copy(lse_v.at[pl.ds(0, L)], psum_hbm.at[pl.ds(wid * L, L)])


def kernel(x, targets, W):
    xf = x.reshape(-1)
    tf = targets.reshape(-1)
    w_pad = jnp.pad(W, ((0, VPAD - VOCAB), (0, 0)))
    lse = _row_lse(w_pad)
    wt_pad = jnp.pad(W.T, ((0, 0), (0, VPAD - VOCAB)))
    out_t, psums = _sc_gather(xf, tf, lse, wt_pad)
    loss = jnp.sum(psums) / jnp.float32(N)
    return (jnp.transpose(out_t), loss)


# speedup vs baseline: 3.7861x; 3.7861x over previous
import functools

import jax
import jax.numpy as jnp
from jax import lax
from jax.experimental import pallas as pl
from jax.experimental.pallas import tpu as pltpu
from jax.experimental.pallas import tpu_sc as plsc

VOCAB = 1000
VPAD = 1024
D = 1000
B, T = 1024, 200
N = B * T

NC, NS, L = 2, 16, 16
NW = NC * NS
NBLK = D // 8
SCHUNK = 4096
NSC = N // SCHUNK
B_PER_W = N // NW


def _row_lse_body(w_ref, out_ref):
    w = w_ref[...]
    m = jnp.max(w, axis=1)
    s = jnp.sum(jnp.exp(w - m[:, None]), axis=1)
    out_ref[...] = jnp.log(s) + m


def _row_lse(w_pad):
    return pl.pallas_call(
        _row_lse_body,
        out_shape=jax.ShapeDtypeStruct((VPAD,), jnp.float32),
    )(w_pad)


_MESH = plsc.VectorSubcoreMesh(core_axis_name="c", subcore_axis_name="s")


@functools.partial(
    pl.kernel,
    mesh=_MESH,
    compiler_params=pltpu.CompilerParams(needs_layout_passes=False),
    out_type=[
        jax.ShapeDtypeStruct((D, N), jnp.float32),
        jax.ShapeDtypeStruct((NW * L,), jnp.float32),
    ],
    scratch_types=[
        pltpu.VMEM((B_PER_W,), jnp.int32),
        pltpu.VMEM((VPAD,), jnp.float32),
        pltpu.VMEM((8, VPAD), jnp.float32),
        pltpu.VMEM((2, SCHUNK), jnp.int32),
        pltpu.VMEM((2, SCHUNK), jnp.int32),
        pltpu.VMEM((2, 8, SCHUNK), jnp.float32),
        pltpu.SemaphoreType.DMA,
        pltpu.SemaphoreType.DMA,
        pltpu.SemaphoreType.DMA,
        pltpu.SemaphoreType.DMA,
    ],
)
def _sc_gather(x_hbm, t_hbm, lse_hbm, wt_hbm, out_hbm, psum_hbm,
               idxl_v, lse_v, wt_v, xv_v, tv_v, tbuf_v,
               p0sem, p1sem, w0sem, w1sem):
    wid = lax.axis_index("s") * NC + lax.axis_index("c")
    psem = (p0sem, p1sem)
    wsem = (w0sem, w1sem)
    lanes = lax.iota(jnp.int32, L)

    pltpu.sync_copy(x_hbm.at[pl.ds(wid * B_PER_W, B_PER_W)], idxl_v)
    pltpu.sync_copy(lse_hbm, lse_v)

    def lse_step(k, acc):
        xv = idxl_v[pl.ds(k * L, L)]
        return acc + plsc.load_gather(lse_v, [xv])

    acc_lse = lax.fori_loop(
        0, B_PER_W // L, lse_step, jnp.zeros((L,), jnp.float32)
    )
    acc = (acc_lse, jnp.zeros((L,), jnp.float32))

    def pf(sc, slot, sem):
        off = sc * SCHUNK
        pltpu.async_copy(x_hbm.at[pl.ds(off, SCHUNK)], xv_v.at[slot], sem)
        pltpu.async_copy(t_hbm.at[pl.ds(off, SCHUNK)], tv_v.at[slot], sem)

    pf(0, 0, psem[0])
    pf(1, 1, psem[1])

    nblk_mine = jnp.where(wid < (NBLK % NW), NBLK // NW + 1, NBLK // NW)

    def block_step(bi, acc):
        blk = bi * NW + wid
        pltpu.sync_copy(wt_hbm.at[pl.ds(blk * 8, 8)], wt_v)
        blk8 = jnp.full((L,), 0, jnp.int32) + blk * 8
        rfill = [jnp.full((L,), r, jnp.int32) for r in range(8)]

        def group_step(g, acc2):
            for b in range(2):
                sc = g * 2 + b
                @pl.when(g >= 1)
                def _():
                    pltpu.make_async_copy(
                        tbuf_v.at[b],
                        out_hbm.at[pl.ds(blk * 8, 8), pl.ds(0, SCHUNK)],
                        wsem[b],
                    ).wait()
                pltpu.make_async_copy(
                    x_hbm.at[pl.ds(0, SCHUNK)], xv_v.at[b], psem[b]
                ).wait()
                pltpu.make_async_copy(
                    t_hbm.at[pl.ds(0, SCHUNK)], tv_v.at[b], psem[b]
                ).wait()

                @plsc.parallel_loop(0, SCHUNK // L, 1, unroll=4, carry=acc2)
                def k_loop(kk, acc3):
                    a0, a1 = acc3
                    xvv = xv_v[b, pl.ds(kk * L, L)]
                    tvv = tv_v[b, pl.ds(kk * L, L)]
                    for r in range(8):
                        v = plsc.load_gather(wt_v, [rfill[r], xvv])
                        tbuf_v[b, r, pl.ds(kk * L, L)] = v
                    tloc = tvv - blk8
                    m = (tloc >= 0) & (tloc < 8)
                    tcl = jnp.clip(tloc, 0, 7)
                    p = plsc.load_gather(wt_v, [tcl, xvv])
                    return (a0 - jnp.where(m, p, 0.0), a1)

                acc2 = k_loop
                pltpu.async_copy(
                    tbuf_v.at[b],
                    out_hbm.at[pl.ds(blk * 8, 8), pl.ds(sc * SCHUNK, SCHUNK)],
                    wsem[b],
                )
                pf(lax.rem(sc + 2, NSC), b, psem[b])
            return acc2

        acc = lax.fori_loop(0, NSC // 2, group_step, acc)
        for b in range(2):
            pltpu.make_async_copy(
                tbuf_v.at[b],
                out_hbm.at[pl.ds(blk * 8, 8), pl.ds(0, SCHUNK)],
                wsem[b],
            ).wait()
        return acc

    acc = lax.fori_loop(0, nblk_mine, block_step, acc)

    for b in range(2):
        pltpu.make_async_copy(
            x_hbm.at[pl.ds(0, SCHUNK)], xv_v.at[b], psem[b]
        ).wait()
        pltpu.make_async_copy(
            t_hbm.at[pl.ds(0, SCHUNK)], tv_v.at[b], psem[b]
        ).wait()
    lse_v[pl.ds(0, L)] = acc[0] + acc[1]
    pltpu.sync_copy(lse_v.at[pl.ds(0, L)], psum_hbm.at[pl.ds(wid * L, L)])


def kernel(x, targets, W):
    xf = x.reshape(-1)
    tf = targets.reshape(-1)
    w_pad = jnp.pad(W, ((0, VPAD - VOCAB), (0, 0)))
    lse = _row_lse(w_pad)
    wt_pad = jnp.pad(W.T, ((0, 0), (0, VPAD - VOCAB)))
    out_t, psums = _sc_gather(xf, tf, lse, wt_pad)
    loss = jnp.sum(psums) / jnp.float32(N)
    return (jnp.transpose(out_t), loss)
